# speedup vs baseline: 2.9037x; 2.9037x over previous
import functools

import jax
import jax.numpy as jnp
from jax import lax
from jax.experimental import pallas as pl
from jax.experimental.pallas import tpu as pltpu
from jax.experimental.pallas import tpu_sc as plsc

N = 100000
VOCAB = 513
D = 128
L = 16
NC = 2
NS = 16
NW = NC * NS
C = 400
NUM_CHUNKS = N // C
MAXC = (NUM_CHUNKS + NW - 1) // NW

_mesh = plsc.VectorSubcoreMesh(core_axis_name="c", subcore_axis_name="s")


@functools.partial(
    pl.kernel,
    mesh=_mesh,
    out_type=jax.ShapeDtypeStruct((N, D), jnp.float32),
    scratch_types=[
        pltpu.VMEM((C,), jnp.int32),
        pltpu.VMEM((C,), jnp.int32),
        pltpu.VMEM((C, D), jnp.float32),
        pltpu.VMEM((C, D), jnp.float32),
        pltpu.SemaphoreType.DMA,
    ],
)
def _degree_encode(ind_hbm, outd_hbm, enc1_hbm, enc2_hbm, out_hbm,
                   idx1_v, idx2_v, rows1_v, rows2_v, sem):
    w = lax.axis_index("s") * NC + lax.axis_index("c")

    def chunk_body(i, carry):
        chunk = i * NW + w

        @pl.when(chunk < NUM_CHUNKS)
        def _():
            base = chunk * C
            pltpu.sync_copy(ind_hbm.at[pl.ds(base, C)], idx1_v)
            pltpu.sync_copy(outd_hbm.at[pl.ds(base, C)], idx2_v)

            def clamp_body(j, cc):
                s = pl.ds(j * L, L)
                idx1_v[s] = jnp.clip(idx1_v[s], 0, VOCAB - 1)
                idx2_v[s] = jnp.clip(idx2_v[s], 0, VOCAB - 1)
                return cc

            lax.fori_loop(0, C // L, clamp_body, 0)

            cp1 = pltpu.async_copy(enc1_hbm.at[idx1_v], rows1_v, sem)
            cp2 = pltpu.async_copy(enc2_hbm.at[idx2_v], rows2_v, sem)
            cp1.wait()
            cp2.wait()

            def add_body(r, cc):
                for k in range(D // L):
                    s = pl.ds(k * L, L)
                    rows1_v[r, s] = rows1_v[r, s] + rows2_v[r, s]
                return cc

            lax.fori_loop(0, C, add_body, 0)

            pltpu.sync_copy(rows1_v, out_hbm.at[pl.ds(base, C)])

        return carry

    lax.fori_loop(0, MAXC, chunk_body, 0)


def kernel(in_degree, out_degree, enc1, enc2):
    ind = in_degree.astype(jnp.int32)
    outd = out_degree.astype(jnp.int32)
    return _degree_encode(ind, outd, enc1, enc2)


# speedup vs baseline: 3.2131x; 1.1066x over previous
import functools

import jax
import jax.numpy as jnp
from jax import lax
from jax.experimental import pallas as pl
from jax.experimental.pallas import tpu as pltpu
from jax.experimental.pallas import tpu_sc as plsc

N = 100000
VOCAB = 513
D = 128
L = 16
NC = 2
NS = 16
NW = NC * NS
C = 200
NUM_CHUNKS = N // C
MAXC = (NUM_CHUNKS + NW - 1) // NW

_CLAMP_OFFS = list(range(0, C - L + 1, L))
if C % L:
    _CLAMP_OFFS.append(C - L)

_mesh = plsc.VectorSubcoreMesh(core_axis_name="c", subcore_axis_name="s")


@functools.partial(
    pl.kernel,
    mesh=_mesh,
    out_type=jax.ShapeDtypeStruct((N, D), jnp.float32),
    scratch_types=[
        pltpu.VMEM((C,), jnp.int32),
        pltpu.VMEM((C,), jnp.int32),
        pltpu.VMEM((C,), jnp.int32),
        pltpu.VMEM((C,), jnp.int32),
        pltpu.VMEM((C, D), jnp.float32),
        pltpu.VMEM((C, D), jnp.float32),
        pltpu.VMEM((C, D), jnp.float32),
        pltpu.VMEM((C, D), jnp.float32),
        pltpu.SemaphoreType.DMA,
        pltpu.SemaphoreType.DMA,
        pltpu.SemaphoreType.DMA,
    ],
)
def _degree_encode(ind_hbm, outd_hbm, enc1_hbm, enc2_hbm, out_hbm,
                   idx1a, idx1b, idx2a, idx2b,
                   rows1a, rows1b, rows2a, rows2b,
                   sem_i, sem_g, sem_wb):
    w = lax.axis_index("s") * NC + lax.axis_index("c")
    idx1 = (idx1a, idx1b)
    idx2 = (idx2a, idx2b)
    rows1 = (rows1a, rows1b)
    rows2 = (rows2a, rows2b)

    def valid(t):
        return (t * NW + w) < NUM_CHUNKS

    def base(t):
        return (t * NW + w) * C

    def idx_descs(t, b):
        s = pl.ds(base(t), C)
        return (pltpu.make_async_copy(ind_hbm.at[s], idx1[b], sem_i),
                pltpu.make_async_copy(outd_hbm.at[s], idx2[b], sem_i))

    def gather_descs(b):
        return (pltpu.make_async_copy(enc1_hbm.at[idx1[b]], rows1[b], sem_g),
                pltpu.make_async_copy(enc2_hbm.at[idx2[b]], rows2[b], sem_g))

    def wb_desc(t, b):
        return pltpu.make_async_copy(rows1[b], out_hbm.at[pl.ds(base(t), C)],
                                     sem_wb)

    def clamp(b):
        for off in _CLAMP_OFFS:
            s = pl.ds(off, L)
            idx1[b][s] = jnp.clip(idx1[b][s], 0, VOCAB - 1)
            idx2[b][s] = jnp.clip(idx2[b][s], 0, VOCAB - 1)

    for d in idx_descs(0, 0):
        d.start()
    for d in idx_descs(0, 0):
        d.wait()
    clamp(0)
    for d in gather_descs(0):
        d.start()

    if NW < NUM_CHUNKS:
        @pl.when(valid(1))
        def _():
            for d in idx_descs(1, 1):
                d.start()

    for t in range(MAXC):
        if t * NW >= NUM_CHUNKS:
            break
        p = t % 2
        q = 1 - p

        @pl.when(valid(t))
        def _():
            for d in gather_descs(p):
                d.wait()

        if (t + 1) * NW < NUM_CHUNKS:
            @pl.when(valid(t + 1))
            def _():
                for d in idx_descs(t + 1, q):
                    d.wait()
                clamp(q)
                if t >= 1:
                    wb_desc(t - 1, q).wait()
                for d in gather_descs(q):
                    d.start()

        if (t + 2) * NW < NUM_CHUNKS:
            @pl.when(valid(t + 2))
            def _():
                for d in idx_descs(t + 2, p):
                    d.start()

        @pl.when(valid(t))
        def _():
            def add_body(r, cc):
                for k in range(D // L):
                    s = pl.ds(k * L, L)
                    rows1[p][r, s] = rows1[p][r, s] + rows2[p][r, s]
                return cc

            lax.fori_loop(0, C, add_body, 0)
            wb_desc(t, p).start()

    wb_desc(0, 0).wait()
    wb_desc(0, 1).wait()


def kernel(in_degree, out_degree, enc1, enc2):
    ind = in_degree.astype(jnp.int32)
    outd = out_degree.astype(jnp.int32)
    return _degree_encode(ind, outd, enc1, enc2)


# speedup vs baseline: 5.9386x; 1.8482x over previous
import functools

import jax
import jax.numpy as jnp
from jax import lax
from jax.experimental import pallas as pl
from jax.experimental.pallas import tpu as pltpu
from jax.experimental.pallas import tpu_sc as plsc

N = 100000
VOCAB = 513
D = 128
L = 16
NC = 2
NS = 16
NW = NC * NS
C = 200
NUM_CHUNKS = N // C
MAXC = (NUM_CHUNKS + NW - 1) // NW
VPT = VOCAB // NS

_CLAMP_OFFS = list(range(0, C - L + 1, L))
if C % L:
    _CLAMP_OFFS.append(C - L)

_mesh = plsc.VectorSubcoreMesh(core_axis_name="c", subcore_axis_name="s")


@functools.partial(
    pl.kernel,
    mesh=_mesh,
    out_type=jax.ShapeDtypeStruct((N, D), jnp.float32),
    scratch_types=[
        pltpu.VMEM((C,), jnp.int32),
        pltpu.VMEM((C,), jnp.int32),
        pltpu.VMEM((C,), jnp.int32),
        pltpu.VMEM((C,), jnp.int32),
        pltpu.VMEM((C, D), jnp.float32),
        pltpu.VMEM((C, D), jnp.float32),
        pltpu.VMEM((C, D), jnp.float32),
        pltpu.VMEM((C, D), jnp.float32),
        pltpu.VMEM_SHARED((VOCAB, D), jnp.float32),
        pltpu.VMEM_SHARED((VOCAB, D), jnp.float32),
        pltpu.SemaphoreType.DMA,
        pltpu.SemaphoreType.DMA,
        pltpu.SemaphoreType.DMA,
    ],
)
def _degree_encode(ind_hbm, outd_hbm, enc1_hbm, enc2_hbm, out_hbm,
                   idx1a, idx1b, idx2a, idx2b,
                   rows1a, rows1b, rows2a, rows2b,
                   sh1, sh2,
                   sem_i, sem_g, sem_wb):
    cid = lax.axis_index("c")
    sid = lax.axis_index("s")
    w = sid * NC + cid
    idx1 = (idx1a, idx1b)
    idx2 = (idx2a, idx2b)
    rows1 = (rows1a, rows1b)
    rows2 = (rows2a, rows2b)

    r0 = sid * VPT
    pltpu.sync_copy(enc1_hbm.at[pl.ds(r0, VPT)], sh1.at[pl.ds(r0, VPT)])
    pltpu.sync_copy(enc2_hbm.at[pl.ds(r0, VPT)], sh2.at[pl.ds(r0, VPT)])

    @pl.when(sid == 0)
    def _():
        tail = pl.ds(NS * VPT, VOCAB - NS * VPT)
        pltpu.sync_copy(enc1_hbm.at[tail], sh1.at[tail])
        pltpu.sync_copy(enc2_hbm.at[tail], sh2.at[tail])

    plsc.subcore_barrier()

    def valid(t):
        return (t * NW + w) < NUM_CHUNKS

    def base(t):
        return (t * NW + w) * C

    def idx_descs(t, b):
        s = pl.ds(base(t), C)
        return (pltpu.make_async_copy(ind_hbm.at[s], idx1[b], sem_i),
                pltpu.make_async_copy(outd_hbm.at[s], idx2[b], sem_i))

    def gather_descs(b):
        return (pltpu.make_async_copy(sh1.at[idx1[b]], rows1[b], sem_g),
                pltpu.make_async_copy(sh2.at[idx2[b]], rows2[b], sem_g))

    def wb_desc(t, b):
        return pltpu.make_async_copy(rows1[b], out_hbm.at[pl.ds(base(t), C)],
                                     sem_wb)

    def clamp(b):
        for off in _CLAMP_OFFS:
            s = pl.ds(off, L)
            idx1[b][s] = jnp.clip(idx1[b][s], 0, VOCAB - 1)
            idx2[b][s] = jnp.clip(idx2[b][s], 0, VOCAB - 1)

    for d in idx_descs(0, 0):
        d.start()
    for d in idx_descs(0, 0):
        d.wait()
    clamp(0)
    for d in gather_descs(0):
        d.start()

    if NW < NUM_CHUNKS:
        @pl.when(valid(1))
        def _():
            for d in idx_descs(1, 1):
                d.start()

    for t in range(MAXC):
        if t * NW >= NUM_CHUNKS:
            break
        p = t % 2
        q = 1 - p

        @pl.when(valid(t))
        def _():
            for d in gather_descs(p):
                d.wait()

        if (t + 1) * NW < NUM_CHUNKS:
            @pl.when(valid(t + 1))
            def _():
                for d in idx_descs(t + 1, q):
                    d.wait()
                clamp(q)
                if t >= 1:
                    wb_desc(t - 1, q).wait()
                for d in gather_descs(q):
                    d.start()

        if (t + 2) * NW < NUM_CHUNKS:
            @pl.when(valid(t + 2))
            def _():
                for d in idx_descs(t + 2, p):
                    d.start()

        @pl.when(valid(t))
        def _():
            def add_body(r, cc):
                for k in range(D // L):
                    s = pl.ds(k * L, L)
                    rows1[p][r, s] = rows1[p][r, s] + rows2[p][r, s]
                return cc

            lax.fori_loop(0, C, add_body, 0)
            wb_desc(t, p).start()

    wb_desc(0, 0).wait()
    wb_desc(0, 1).wait()


def kernel(in_degree, out_degree, enc1, enc2):
    ind = in_degree.astype(jnp.int32)
    outd = out_degree.astype(jnp.int32)
    return _degree_encode(ind, outd, enc1, enc2)
